# Initial kernel scaffold; baseline (speedup 1.0000x reference)
import jax
import jax.numpy as jnp
from jax.experimental import pallas as pl


def kernel(x, edge_index, W_l, W_r, b):
    raise NotImplementedError("write your pallas kernel here")


# speedup vs baseline: 12.8606x; 12.8606x over previous
import functools

import jax
import jax.numpy as jnp
from jax import lax
from jax.experimental import pallas as pl
from jax.experimental.pallas import tpu as pltpu
from jax.experimental.pallas import tpu_sc as plsc

NC = 2
NS = 16
NW = NC * NS
K = 128


def _mm_body(x_ref, wl_ref, wr_ref, b_ref, yl_ref, yr_ref):
    xb = x_ref[...]
    dn = (((1,), (1,)), ((), ()))
    yl_ref[...] = lax.dot_general(xb, wl_ref[...], dn,
                                  preferred_element_type=jnp.float32)
    yr_ref[...] = lax.dot_general(xb, wr_ref[...], dn,
                                  preferred_element_type=jnp.float32) + b_ref[...]


def _fin_body(p_ref, c_ref, yr_ref, o_ref):
    seg = p_ref[0] + p_ref[1]
    o_ref[...] = seg / jnp.maximum(c_ref[...], 1.0) + yr_ref[...]


def _make_sc_kernel(n_pad, ch, d):
    rpt = n_pad // NS

    mesh = plsc.VectorSubcoreMesh(core_axis_name="c", subcore_axis_name="s")

    @functools.partial(
        pl.kernel,
        out_type=[
            jax.ShapeDtypeStruct((NC, n_pad, d), jnp.float32),
            jax.ShapeDtypeStruct((NC, 1, n_pad), jnp.float32),
        ],
        mesh=mesh,
        scratch_types=[
            pltpu.VMEM((ch // 2, K), jnp.int32),
            pltpu.VMEM((ch // 2, K), jnp.int32),
            pltpu.VMEM((K, d), jnp.float32),
            pltpu.VMEM((K, d), jnp.float32),
            pltpu.VMEM((K,), jnp.float32),
            pltpu.VMEM_SHARED((n_pad, d), jnp.float32),
            pltpu.VMEM_SHARED((n_pad,), jnp.float32),
            pltpu.SemaphoreType.DMA,
            pltpu.SemaphoreType.DMA,
            pltpu.SemaphoreType.DMA,
            pltpu.SemaphoreType.DMA,
            pltpu.SemaphoreType.DMA,
            pltpu.SemaphoreType.DMA,
        ],
    )
    def sc_kernel(yl_hbm, src_hbm, dst_hbm, zer_d_hbm, zer_1_hbm, one_hbm,
                  acc_out, cnt_out, srci, dsti, bufa, bufb, ones_v,
                  acc_sh, cnt_sh, sga, sgb, ssa, ssb, sca, scb):
        c = lax.axis_index("c")
        s = lax.axis_index("s")
        wid = s * NC + c
        pltpu.sync_copy(one_hbm, ones_v)
        pltpu.sync_copy(zer_d_hbm, acc_sh.at[pl.ds(s * rpt, rpt)])
        pltpu.sync_copy(zer_1_hbm, cnt_sh.at[pl.ds(s * rpt, rpt)])
        plsc.subcore_barrier()

        hc = ch // 2
        bufs = (bufa, bufb)
        gsems = (sga, sgb)
        ssems = (ssa, ssb)
        csems = (sca, scb)
        for half in range(2):
            hs = half * hc
            pltpu.sync_copy(src_hbm.at[wid, pl.ds(hs, hc)], srci)
            pltpu.sync_copy(dst_hbm.at[wid, pl.ds(hs, hc)], dsti)
            gd = [None] * hc
            sd = [None] * hc
            cd = [None] * hc
            gd[0] = pltpu.async_copy(yl_hbm.at[srci.at[0]], bufs[0], gsems[0])
            for g in range(hc):
                k = g & 1
                if g >= 1:
                    sd[g - 1].wait()
                    cd[g - 1].wait()
                if g + 1 < hc:
                    gd[g + 1] = pltpu.async_copy(
                        yl_hbm.at[srci.at[g + 1]], bufs[1 - k], gsems[1 - k])
                gd[g].wait()
                sd[g] = pltpu.async_copy(bufs[k], acc_sh.at[dsti.at[g]],
                                         ssems[k], add=True)
                cd[g] = pltpu.async_copy(ones_v, cnt_sh.at[dsti.at[g]],
                                         csems[k], add=True)
            sd[hc - 1].wait()
            cd[hc - 1].wait()

        plsc.subcore_barrier()
        pltpu.sync_copy(acc_sh.at[pl.ds(s * rpt, rpt)],
                        acc_out.at[c, pl.ds(s * rpt, rpt)])
        pltpu.sync_copy(cnt_sh.at[pl.ds(s * rpt, rpt)],
                        cnt_out.at[c, 0, pl.ds(s * rpt, rpt)])

    return sc_kernel


def kernel(x, edge_index, W_l, W_r, b):
    n, d = x.shape
    e = edge_index.shape[1]

    ch = -(--(-e // (NW * K)) // 4) * 4
    ew = ch * K
    e_pad = ew * NW
    n_pad = -(-(n + 1) // (NS * K)) * (NS * K)
    rpt = n_pad // NS

    src = edge_index[0].astype(jnp.int32)
    dst = edge_index[1].astype(jnp.int32)
    pad = e_pad - e
    pad_i = jnp.arange(pad, dtype=jnp.int32)
    src = jnp.concatenate([src, pad_i % n]).reshape(NW, ch, K)
    dst = jnp.concatenate([dst, n + pad_i % (n_pad - n)]).reshape(NW, ch, K)

    zer_d = jnp.zeros((rpt, d), jnp.float32)
    zer_1 = jnp.zeros((rpt,), jnp.float32)
    one = jnp.ones((K,), jnp.float32)

    rb = 1000
    grid = (n // rb,)
    yl, yr = pl.pallas_call(
        _mm_body,
        grid=grid,
        in_specs=[
            pl.BlockSpec((rb, d), lambda i: (i, 0)),
            pl.BlockSpec((d, d), lambda i: (0, 0)),
            pl.BlockSpec((d, d), lambda i: (0, 0)),
            pl.BlockSpec((1, d), lambda i: (0, 0)),
        ],
        out_specs=[
            pl.BlockSpec((rb, d), lambda i: (i, 0)),
            pl.BlockSpec((rb, d), lambda i: (i, 0)),
        ],
        out_shape=[
            jax.ShapeDtypeStruct((n, d), jnp.float32),
            jax.ShapeDtypeStruct((n, d), jnp.float32),
        ],
    )(x, W_l, W_r, b.reshape(1, d))

    parts, cnts = _make_sc_kernel(n_pad, ch, d)(yl, src, dst, zer_d, zer_1, one)

    out = pl.pallas_call(
        _fin_body,
        grid=grid,
        in_specs=[
            pl.BlockSpec((NC, rb, d), lambda i: (0, i, 0)),
            pl.BlockSpec((rb, 1), lambda i: (i, 0)),
            pl.BlockSpec((rb, d), lambda i: (i, 0)),
        ],
        out_specs=pl.BlockSpec((rb, d), lambda i: (i, 0)),
        out_shape=jax.ShapeDtypeStruct((n, d), jnp.float32),
    )(parts, (cnts[0, 0] + cnts[1, 0])[:n].reshape(n, 1), yr)
    return out
